# baseline (speedup 1.0000x reference)
import functools

import jax
import jax.numpy as jnp
from jax import lax
from jax.experimental import pallas as pl
from jax.experimental.pallas import tpu as pltpu
from jax.experimental.pallas import tpu_sc as plsc

H = 128
EE = 32
WIDE = H + EE
B, NP, ND = 8, 2048, 128
K = 16
DEPTH = 3

NC, NS = 2, 16
NW = NC * NS
PPT = (B * NP) // NW
DPT = (B * ND) // NW
CH = 8
INV_SQRT_H = 1.0 / float(H) ** 0.5
NF = H // 16


def _fullspec(shape):
    nd = len(shape)
    return pl.BlockSpec(shape, lambda b, _n=nd: (0,) * _n)


def _bspec(shape):
    nd = len(shape)
    return pl.BlockSpec((1,) + shape, lambda b, _n=nd: (b,) + (0,) * _n)


def _mm(x, w):
    return jnp.dot(x, w, preferred_element_type=jnp.float32)


def _mmT(x, w):
    return lax.dot_general(x, w, (((1,), (1,)), ((), ())),
                           preferred_element_type=jnp.float32)


def _stage0_body(pf, df, er2a, ea2r,
                 Wp1, bp1, Wp2, bp2, Wd1, bd1, Wd2, bd2,
                 Wpb, bpb, Wdb, bdb, Wpce, bpc, Wdce, bdc,
                 p0_o, d0_o, Tp_o, Td_o):
    p0 = _mm(jax.nn.relu(_mm(pf[0], Wp1[...]) + bp1[...]), Wp2[...]) + bp2[...]
    d0 = _mm(jax.nn.relu(_mm(df[0], Wd1[...]) + bd1[...]), Wd2[...]) + bd2[...]
    p0_o[0] = p0
    d0_o[0] = d0
    ep = _mm(ea2r[0], Wpb[...]) + bpb[...]
    ed = _mm(er2a[0], Wdb[...]) + bdb[...]
    for l in range(DEPTH):
        Tp_o[l, 0, :, 0:H] = _mm(ep, Wpce[l]) + bpc[l]
        Tp_o[l, 0, :, H:WIDE] = ep
        Td_o[l, 0, :, 0:H] = _mm(ed, Wdce[l]) + bdc[l]
        Td_o[l, 0, :, H:WIDE] = ed


def _stage0(pf, df, er2a, ea2r, w):
    out_shape = (
        jax.ShapeDtypeStruct((B, NP, H), jnp.float32),
        jax.ShapeDtypeStruct((B, ND, H), jnp.float32),
        jax.ShapeDtypeStruct((DEPTH, B, NP, WIDE), jnp.float32),
        jax.ShapeDtypeStruct((DEPTH, B, ND, WIDE), jnp.float32),
    )
    ins = (pf, df, er2a, ea2r,
           w["Wu_prot1"]["W"], w["Wu_prot1"]["b"], w["Wu_prot2"]["W"], w["Wu_prot2"]["b"],
           w["Wu_drug1"]["W"], w["Wu_drug1"]["b"], w["Wu_drug2"]["W"], w["Wu_drug2"]["b"],
           w["pb_emb"]["W"], w["pb_emb"]["b"], w["db_emb"]["W"], w["db_emb"]["b"],
           jnp.stack([L["Wpc"]["W"][H:] for L in w["layers"]]),
           jnp.stack([L["Wpc"]["b"] for L in w["layers"]]),
           jnp.stack([L["Wdc"]["W"][H:] for L in w["layers"]]),
           jnp.stack([L["Wdc"]["b"] for L in w["layers"]]))
    in_specs = [_bspec(x.shape[1:]) for x in (pf, df, er2a, ea2r)]
    in_specs += [_fullspec(x.shape) for x in ins[4:]]
    out_specs = (
        _bspec((NP, H)), _bspec((ND, H)),
        pl.BlockSpec((DEPTH, 1, NP, WIDE), lambda b: (0, b, 0, 0)),
        pl.BlockSpec((DEPTH, 1, ND, WIDE), lambda b: (0, b, 0, 0)),
    )
    return pl.pallas_call(
        _stage0_body, grid=(B,), out_shape=out_shape,
        in_specs=in_specs, out_specs=out_specs)(*ins)


def _pre_body(up, ud, Wdcn, Wpcn, WQp, bQp, WKp, WQd, bQd, WKd,
              Gd_o, qkp_o, Sp_o, qkd_o):
    u = up[0]
    v = ud[0]
    Gd_o[0, :, 0:H] = _mm(u, Wdcn[...])
    Gd_o[0, :, H:WIDE] = jnp.zeros((NP, EE), jnp.float32)
    Sp_o[0] = _mm(v, Wpcn[...])
    qkp_o[0] = _mmT(_mm(u, WQp[...]) + bQp[...], WKp[...])
    qkd_o[0] = _mmT(_mm(v, WQd[...]) + bQd[...], WKd[...])


def _pre_tc(up, ud, L):
    out_shape = (
        jax.ShapeDtypeStruct((B, NP, WIDE), jnp.float32),
        jax.ShapeDtypeStruct((B, NP, H), jnp.float32),
        jax.ShapeDtypeStruct((B, ND, H), jnp.float32),
        jax.ShapeDtypeStruct((B, ND, H), jnp.float32),
    )
    ins = (up, ud, L["Wdc"]["W"][:H], L["Wpc"]["W"][:H],
           L["WQp"]["W"], L["WQp"]["b"], L["WKp"]["W"],
           L["WQd"]["W"], L["WQd"]["b"], L["WKd"]["W"])
    in_specs = [_bspec((NP, H)), _bspec((ND, H))]
    in_specs += [_fullspec(x.shape) for x in ins[2:]]
    out_specs = (_bspec((NP, WIDE)), _bspec((NP, H)), _bspec((ND, H)), _bspec((ND, H)))
    return pl.pallas_call(
        _pre_body, grid=(B,), out_shape=out_shape,
        in_specs=in_specs, out_specs=out_specs)(*ins)


def _post_body(Op, Od, up, ud, p0, d0,
               Wp, bp, Wpup, bpup, Wd, bd, Wdup, bdup,
               up_o, ud_o, *, add_res):
    hp = jax.nn.relu(_mm(Op[0], Wp[...]) + bp[...])
    hd = jax.nn.relu(_mm(Od[0], Wd[...]) + bd[...])
    un = up[0] + _mm(hp, Wpup[...]) + bpup[...]
    vn = ud[0] + _mm(hd, Wdup[...]) + bdup[...]
    if add_res:
        un = un + p0[0]
        vn = vn + d0[0]
    up_o[0] = un
    ud_o[0] = vn


def _post_tc(Op, Od, up, ud, p0, d0, L, add_res):
    out_shape = (
        jax.ShapeDtypeStruct((B, NP, H), jnp.float32),
        jax.ShapeDtypeStruct((B, ND, H), jnp.float32),
    )
    ins = (Op, Od, up, ud, p0, d0,
           L["Wp"]["W"], L["Wp"]["b"], L["Wpup"]["W"], L["Wpup"]["b"],
           L["Wd"]["W"], L["Wd"]["b"], L["Wdup"]["W"], L["Wdup"]["b"])
    in_specs = [_bspec((NP, WIDE)), _bspec((ND, WIDE)),
                _bspec((NP, H)), _bspec((ND, H)),
                _bspec((NP, H)), _bspec((ND, H))]
    in_specs += [_fullspec(x.shape) for x in ins[6:]]
    out_specs = (_bspec((NP, H)), _bspec((ND, H)))
    return pl.pallas_call(
        functools.partial(_post_body, add_res=add_res),
        grid=(B,), out_shape=out_shape,
        in_specs=in_specs, out_specs=out_specs)(*ins)


def _attend_node(n, sl, qk_v, out_v, idx_row, nei_ld, eproj_ld, eraw_ld):
    qk = [qk_v[sl, n, pl.ds(16 * f, 16)] for f in range(NF)]
    accs = [jnp.zeros((16,), jnp.float32) for _ in range(NF + 2)]
    dsum = jnp.zeros((16,), jnp.float32)
    for k in range(K):
        ctx = (idx_row[k], n * K + k)
        ms = []
        acc = None
        for f in range(NF):
            x = nei_ld(ctx, f) + eproj_ld(ctx, f)
            m = jnp.maximum(x, 0.1 * x)
            ms.append(m)
            t = qk[f] * m
            acc = t if acc is None else acc + t
        s = jnp.sum(acc) * INV_SQRT_H
        ev = jnp.exp(jnp.zeros((16,), jnp.float32) + s)
        dsum = dsum + ev
        for f in range(NF):
            accs[f] = accs[f] + ev * ms[f]
        for f in range(2):
            accs[NF + f] = accs[NF + f] + ev * eraw_ld(ctx, f)
    inv = 1.0 / (dsum + 1e-6)
    for f in range(NF + 2):
        out_v[sl, n, pl.ds(16 * f, 16)] = accs[f] * inv


def _pipeline(nch, fetch_start, fetch_wait, out_start, out_wait, compute):
    fetch_start(0, 0)

    def step(c, carry):
        s = c % 2
        for t in (0, 1):
            @pl.when(s == t)
            def _(t=t):
                @pl.when(c + 1 < nch)
                def _():
                    fetch_start(c + 1, 1 - t)
                fetch_wait(c, t)

                @pl.when(c >= 2)
                def _():
                    out_wait(c - 2, t)
        compute(c, s)
        for t in (0, 1):
            @pl.when(s == t)
            def _(t=t):
                out_start(c, t)
        return carry

    lax.fori_loop(0, nch, step, 0)
    for c in (nch - 2, nch - 1):
        out_wait(c, c % 2)


def _sc_body(Tp, Gd, Sp, Td, qkp, qkd, pn, pb, dn, db, Op, Od,
             small_p, small_d, pn_v, pb_v, dn_v, db_v,
             buf, qk_v, out_v, sg0, sg1, sq0, sq1, so0, so1):
    cid = lax.axis_index("c")
    sid = lax.axis_index("s")
    wid = sid * NC + cid
    bat = wid // (NW // B)
    sg = (sg0, sg1)
    sq = (sq0, sq1)
    so = (so0, so1)

    pltpu.sync_copy(Sp.at[bat], small_p)
    pltpu.sync_copy(Td.at[bat], small_d)
    pltpu.sync_copy(pn.at[pl.ds(wid * PPT, PPT)], pn_v)
    pltpu.sync_copy(pb.at[pl.ds(wid * PPT * K, PPT * K)], pb_v)
    pltpu.sync_copy(dn.at[pl.ds(wid * DPT * K, DPT * K)], dn_v)
    pltpu.sync_copy(db.at[pl.ds(wid * DPT, DPT)], db_v)

    def make_phase(tab, qk_hbm, out_hbm, idx_v, per_tile, is_protein, unroll=1):
        def g_copy(c, t):
            return pltpu.make_async_copy(
                tab.at[idx_v.at[pl.ds(c * CH * K, CH * K)]], buf.at[t], sg[t])

        def q_copy(c, t):
            gbase = wid * per_tile + c * CH
            return pltpu.make_async_copy(
                qk_hbm.at[pl.ds(gbase, CH)], qk_v.at[t], sq[t])

        def o_copy(c, t):
            gbase = wid * per_tile + c * CH
            return pltpu.make_async_copy(
                out_v.at[t], out_hbm.at[pl.ds(gbase, CH)], so[t])

        def fetch_start(c, t):
            g_copy(c, t).start()
            q_copy(c, t).start()

        def fetch_wait(c, t):
            g_copy(c, t).wait()
            q_copy(c, t).wait()

        def out_start(c, t):
            o_copy(c, t).start()

        def out_wait(c, t):
            o_copy(c, t).wait()

        def compute(c, s):
            @plsc.parallel_loop(0, CH, unroll=unroll)
            def node(n):
                nl = c * CH + n
                if is_protein:
                    _attend_node(
                        n, s, qk_v, out_v, pn_v[nl, :],
                        lambda ctx, f: small_p[ctx[0], pl.ds(16 * f, 16)],
                        lambda ctx, f: buf[s, ctx[1], pl.ds(16 * f, 16)],
                        lambda ctx, f: buf[s, ctx[1], pl.ds(H + 16 * f, 16)])
                else:
                    _attend_node(
                        n, s, qk_v, out_v, db_v[nl, :],
                        lambda ctx, f: buf[s, ctx[1], pl.ds(16 * f, 16)],
                        lambda ctx, f: small_d[ctx[0], pl.ds(16 * f, 16)],
                        lambda ctx, f: small_d[ctx[0], pl.ds(H + 16 * f, 16)])

        return fetch_start, fetch_wait, out_start, out_wait, compute

    _pipeline(PPT // CH, *make_phase(Tp, qkp, Op, pb_v, PPT, True, unroll=2))
    _pipeline(DPT // CH, *make_phase(Gd, qkd, Od, dn_v, DPT, False))


def _sc_layer(Tp_f, Gd_f, Sp, Td, qkp_f, qkd_f, pn_f, pb_f, dn_f, db_f):
    mesh = plsc.VectorSubcoreMesh(core_axis_name="c", subcore_axis_name="s",
                                  num_cores=NC, num_subcores=NS)
    out_type = (
        jax.ShapeDtypeStruct((B * NP, WIDE), jnp.float32),
        jax.ShapeDtypeStruct((B * ND, WIDE), jnp.float32),
    )
    scratch = [
        pltpu.VMEM((ND, H), jnp.float32),
        pltpu.VMEM((ND, WIDE), jnp.float32),
        pltpu.VMEM((PPT, K), jnp.int32),
        pltpu.VMEM((PPT * K,), jnp.int32),
        pltpu.VMEM((DPT * K,), jnp.int32),
        pltpu.VMEM((DPT, K), jnp.int32),
        pltpu.VMEM((2, CH * K, WIDE), jnp.float32),
        pltpu.VMEM((2, CH, H), jnp.float32),
        pltpu.VMEM((2, CH, WIDE), jnp.float32),
        pltpu.SemaphoreType.DMA,
        pltpu.SemaphoreType.DMA,
        pltpu.SemaphoreType.DMA,
        pltpu.SemaphoreType.DMA,
        pltpu.SemaphoreType.DMA,
        pltpu.SemaphoreType.DMA,
    ]
    f = pl.kernel(_sc_body, out_type, mesh=mesh, scratch_types=scratch,
                  compiler_params=pltpu.CompilerParams(
                      needs_layout_passes=False, use_tc_tiling_on_sc=False))
    return f(Tp_f, Gd_f, Sp, Td, qkp_f, qkd_f, pn_f, pb_f, dn_f, db_f)


def kernel(protein_features, drug_features, edge_features_residue_to_atom,
           edge_features_atom_to_residue, params, protein_neighbors,
           drug_neighbors, protein_bonds, drug_bonds, p_hetero_mask,
           d_hetero_mask):
    w = params
    p0, d0, Tp_all, Td_all = _stage0(
        protein_features, drug_features,
        edge_features_residue_to_atom, edge_features_atom_to_residue, w)

    boff = (jnp.arange(B, dtype=jnp.int32) * NP)[:, None, None]
    pn_f = protein_neighbors.astype(jnp.int32).reshape(B * NP, K)
    pb_f = (protein_bonds.astype(jnp.int32) + boff).reshape(B * NP * K)
    dn_f = (drug_neighbors.astype(jnp.int32) + boff).reshape(B * ND * K)
    db_f = drug_bonds.astype(jnp.int32).reshape(B * ND, K)

    up, ud = p0, d0
    for l in range(DEPTH):
        L = w["layers"][l]
        Gd, qkp, Sp, qkd = _pre_tc(up, ud, L)
        Op_f, Od_f = _sc_layer(
            Tp_all[l].reshape(B * NP, WIDE), Gd.reshape(B * NP, WIDE),
            Sp, Td_all[l],
            qkp.reshape(B * NP, H), qkd.reshape(B * ND, H),
            pn_f, pb_f, dn_f, db_f)
        up, ud = _post_tc(Op_f.reshape(B, NP, WIDE), Od_f.reshape(B, ND, WIDE),
                          up, ud, p0, d0, L, add_res=(l == DEPTH - 1))
    return (up, ud)


# speedup vs baseline: 1.2481x; 1.2481x over previous
import functools

import jax
import jax.numpy as jnp
from jax import lax
from jax.experimental import pallas as pl
from jax.experimental.pallas import tpu as pltpu
from jax.experimental.pallas import tpu_sc as plsc

H = 128
EE = 32
WIDE = H + EE
B, NP, ND = 8, 2048, 128
K = 16
DEPTH = 3

NC, NS = 2, 16
NW = NC * NS
PPT = (B * NP) // NW
DPT = (B * ND) // NW
CH = 8
INV_SQRT_H = 1.0 / float(H) ** 0.5
NF = H // 16


def _fullspec(shape):
    nd = len(shape)
    return pl.BlockSpec(shape, lambda b, _n=nd: (0,) * _n)


def _bspec(shape):
    nd = len(shape)
    return pl.BlockSpec((1,) + shape, lambda b, _n=nd: (b,) + (0,) * _n)


def _mm(x, w):
    return jnp.dot(x, w, preferred_element_type=jnp.float32)


def _mmT(x, w):
    return lax.dot_general(x, w, (((1,), (1,)), ((), ())),
                           preferred_element_type=jnp.float32)


def _stage0_body(pf, df, er2a, ea2r,
                 Wp1, bp1, Wp2, bp2, Wd1, bd1, Wd2, bd2,
                 Wpb, bpb, Wdb, bdb, Wpce, bpc, Wdce, bdc,
                 p0_o, d0_o, Tp_o, Td_o):
    p0 = _mm(jax.nn.relu(_mm(pf[0], Wp1[...]) + bp1[...]), Wp2[...]) + bp2[...]
    d0 = _mm(jax.nn.relu(_mm(df[0], Wd1[...]) + bd1[...]), Wd2[...]) + bd2[...]
    p0_o[0] = p0
    d0_o[0] = d0
    ep = _mm(ea2r[0], Wpb[...]) + bpb[...]
    ed = _mm(er2a[0], Wdb[...]) + bdb[...]
    for l in range(DEPTH):
        Tp_o[l, 0, :, 0:H] = _mm(ep, Wpce[l]) + bpc[l]
        Tp_o[l, 0, :, H:WIDE] = ep
        Td_o[l, 0, :, 0:H] = _mm(ed, Wdce[l]) + bdc[l]
        Td_o[l, 0, :, H:WIDE] = ed


def _stage0(pf, df, er2a, ea2r, w):
    out_shape = (
        jax.ShapeDtypeStruct((B, NP, H), jnp.float32),
        jax.ShapeDtypeStruct((B, ND, H), jnp.float32),
        jax.ShapeDtypeStruct((DEPTH, B, NP, WIDE), jnp.float32),
        jax.ShapeDtypeStruct((DEPTH, B, ND, WIDE), jnp.float32),
    )
    ins = (pf, df, er2a, ea2r,
           w["Wu_prot1"]["W"], w["Wu_prot1"]["b"], w["Wu_prot2"]["W"], w["Wu_prot2"]["b"],
           w["Wu_drug1"]["W"], w["Wu_drug1"]["b"], w["Wu_drug2"]["W"], w["Wu_drug2"]["b"],
           w["pb_emb"]["W"], w["pb_emb"]["b"], w["db_emb"]["W"], w["db_emb"]["b"],
           jnp.stack([L["Wpc"]["W"][H:] for L in w["layers"]]),
           jnp.stack([L["Wpc"]["b"] for L in w["layers"]]),
           jnp.stack([L["Wdc"]["W"][H:] for L in w["layers"]]),
           jnp.stack([L["Wdc"]["b"] for L in w["layers"]]))
    in_specs = [_bspec(x.shape[1:]) for x in (pf, df, er2a, ea2r)]
    in_specs += [_fullspec(x.shape) for x in ins[4:]]
    out_specs = (
        _bspec((NP, H)), _bspec((ND, H)),
        pl.BlockSpec((DEPTH, 1, NP, WIDE), lambda b: (0, b, 0, 0)),
        pl.BlockSpec((DEPTH, 1, ND, WIDE), lambda b: (0, b, 0, 0)),
    )
    return pl.pallas_call(
        _stage0_body, grid=(B,), out_shape=out_shape,
        in_specs=in_specs, out_specs=out_specs)(*ins)


def _pre_body(up, ud, Wdcn, Wpcn, WQp, bQp, WKp, WQd, bQd, WKd,
              Gd_o, qkp_o, Sp_o, qkd_o):
    u = up[0]
    v = ud[0]
    Gd_o[0, :, 0:H] = _mm(u, Wdcn[...])
    Gd_o[0, :, H:WIDE] = jnp.zeros((NP, EE), jnp.float32)
    Sp_o[0] = _mm(v, Wpcn[...])
    qkp_o[0] = _mmT(_mm(u, WQp[...]) + bQp[...], WKp[...])
    qkd_o[0] = _mmT(_mm(v, WQd[...]) + bQd[...], WKd[...])


def _pre_tc(up, ud, L):
    out_shape = (
        jax.ShapeDtypeStruct((B, NP, WIDE), jnp.float32),
        jax.ShapeDtypeStruct((B, NP, H), jnp.float32),
        jax.ShapeDtypeStruct((B, ND, H), jnp.float32),
        jax.ShapeDtypeStruct((B, ND, H), jnp.float32),
    )
    ins = (up, ud, L["Wdc"]["W"][:H], L["Wpc"]["W"][:H],
           L["WQp"]["W"], L["WQp"]["b"], L["WKp"]["W"],
           L["WQd"]["W"], L["WQd"]["b"], L["WKd"]["W"])
    in_specs = [_bspec((NP, H)), _bspec((ND, H))]
    in_specs += [_fullspec(x.shape) for x in ins[2:]]
    out_specs = (_bspec((NP, WIDE)), _bspec((NP, H)), _bspec((ND, H)), _bspec((ND, H)))
    return pl.pallas_call(
        _pre_body, grid=(B,), out_shape=out_shape,
        in_specs=in_specs, out_specs=out_specs)(*ins)


def _post_body(Op, Od, up, ud, p0, d0,
               Wp, bp, Wpup, bpup, Wd, bd, Wdup, bdup,
               up_o, ud_o, *, add_res):
    hp = jax.nn.relu(_mm(Op[0], Wp[...]) + bp[...])
    hd = jax.nn.relu(_mm(Od[0], Wd[...]) + bd[...])
    un = up[0] + _mm(hp, Wpup[...]) + bpup[...]
    vn = ud[0] + _mm(hd, Wdup[...]) + bdup[...]
    if add_res:
        un = un + p0[0]
        vn = vn + d0[0]
    up_o[0] = un
    ud_o[0] = vn


def _post_tc(Op, Od, up, ud, p0, d0, L, add_res):
    out_shape = (
        jax.ShapeDtypeStruct((B, NP, H), jnp.float32),
        jax.ShapeDtypeStruct((B, ND, H), jnp.float32),
    )
    ins = (Op, Od, up, ud, p0, d0,
           L["Wp"]["W"], L["Wp"]["b"], L["Wpup"]["W"], L["Wpup"]["b"],
           L["Wd"]["W"], L["Wd"]["b"], L["Wdup"]["W"], L["Wdup"]["b"])
    in_specs = [_bspec((NP, WIDE)), _bspec((ND, WIDE)),
                _bspec((NP, H)), _bspec((ND, H)),
                _bspec((NP, H)), _bspec((ND, H))]
    in_specs += [_fullspec(x.shape) for x in ins[6:]]
    out_specs = (_bspec((NP, H)), _bspec((ND, H)))
    return pl.pallas_call(
        functools.partial(_post_body, add_res=add_res),
        grid=(B,), out_shape=out_shape,
        in_specs=in_specs, out_specs=out_specs)(*ins)


def _attend_node(n, sl, qk_v, out_v, idx_row, nei_ld, eproj_ld, eraw_ld):
    qk = [qk_v[sl, n, pl.ds(16 * f, 16)] for f in range(NF)]
    accs = [jnp.zeros((16,), jnp.float32) for _ in range(NF + 2)]
    dsum = jnp.zeros((16,), jnp.float32)
    for k in range(K):
        ctx = (idx_row[k], n * K + k)
        ms = []
        acc = None
        for f in range(NF):
            x = nei_ld(ctx, f) + eproj_ld(ctx, f)
            m = jnp.maximum(x, 0.1 * x)
            ms.append(m)
            t = qk[f] * m
            acc = t if acc is None else acc + t
        s = jnp.sum(acc) * INV_SQRT_H
        ev = jnp.exp(jnp.zeros((16,), jnp.float32) + s)
        dsum = dsum + ev
        for f in range(NF):
            accs[f] = accs[f] + ev * ms[f]
        for f in range(2):
            accs[NF + f] = accs[NF + f] + ev * eraw_ld(ctx, f)
    inv = 1.0 / (dsum + 1e-6)
    for f in range(NF + 2):
        out_v[sl, n, pl.ds(16 * f, 16)] = accs[f] * inv


def _pipeline(nch, fetch_start, fetch_wait, out_start, out_wait, compute):
    fetch_start(0, 0)

    def step(c, carry):
        s = c % 2
        for t in (0, 1):
            @pl.when(s == t)
            def _(t=t):
                @pl.when(c + 1 < nch)
                def _():
                    fetch_start(c + 1, 1 - t)
                fetch_wait(c, t)

                @pl.when(c >= 2)
                def _():
                    out_wait(c - 2, t)
        compute(c, s)
        for t in (0, 1):
            @pl.when(s == t)
            def _(t=t):
                out_start(c, t)
        return carry

    lax.fori_loop(0, nch, step, 0)
    for c in (nch - 2, nch - 1):
        out_wait(c, c % 2)


def _sc_body(Tp, Gd, Sp, Td, qkp, qkd, pn, pb, dn, db, Op, Od,
             small_p, small_d, pn_v, pb_v, dn_v, db_v,
             buf, qk_v, out_v, sg0, sg1, sq0, sq1, so0, so1):
    cid = lax.axis_index("c")
    sid = lax.axis_index("s")
    wid = sid * NC + cid
    bat = wid // (NW // B)
    sg = (sg0, sg1)
    sq = (sq0, sq1)
    so = (so0, so1)

    pltpu.sync_copy(Sp.at[bat], small_p)
    pltpu.sync_copy(Td.at[bat], small_d)
    pltpu.sync_copy(pn.at[pl.ds(wid * PPT, PPT)], pn_v)
    pltpu.sync_copy(pb.at[pl.ds(wid * PPT * K, PPT * K)], pb_v)
    pltpu.sync_copy(dn.at[pl.ds(wid * DPT * K, DPT * K)], dn_v)
    pltpu.sync_copy(db.at[pl.ds(wid * DPT, DPT)], db_v)

    def make_phase(tab, qk_hbm, out_hbm, idx_v, per_tile, is_protein, unroll=1):
        def g_copy(c, t):
            return pltpu.make_async_copy(
                tab.at[idx_v.at[pl.ds(c * CH * K, CH * K)]], buf.at[t], sg[t])

        def q_copy(c, t):
            gbase = wid * per_tile + c * CH
            return pltpu.make_async_copy(
                qk_hbm.at[pl.ds(gbase, CH)], qk_v.at[t], sq[t])

        def o_copy(c, t):
            gbase = wid * per_tile + c * CH
            return pltpu.make_async_copy(
                out_v.at[t], out_hbm.at[pl.ds(gbase, CH)], so[t])

        def fetch_start(c, t):
            g_copy(c, t).start()
            q_copy(c, t).start()

        def fetch_wait(c, t):
            g_copy(c, t).wait()
            q_copy(c, t).wait()

        def out_start(c, t):
            o_copy(c, t).start()

        def out_wait(c, t):
            o_copy(c, t).wait()

        def compute(c, s):
            @plsc.parallel_loop(0, CH, unroll=unroll)
            def node(n):
                nl = c * CH + n
                if is_protein:
                    _attend_node(
                        n, s, qk_v, out_v, pn_v[nl, :],
                        lambda ctx, f: small_p[ctx[0], pl.ds(16 * f, 16)],
                        lambda ctx, f: buf[s, ctx[1], pl.ds(16 * f, 16)],
                        lambda ctx, f: buf[s, ctx[1], pl.ds(H + 16 * f, 16)])
                else:
                    _attend_node(
                        n, s, qk_v, out_v, db_v[nl, :],
                        lambda ctx, f: buf[s, ctx[1], pl.ds(16 * f, 16)],
                        lambda ctx, f: small_d[ctx[0], pl.ds(16 * f, 16)],
                        lambda ctx, f: small_d[ctx[0], pl.ds(H + 16 * f, 16)])

        return fetch_start, fetch_wait, out_start, out_wait, compute

    _pipeline(PPT // CH, *make_phase(Tp, qkp, Op, pb_v, PPT, True, unroll=1))
    _pipeline(DPT // CH, *make_phase(Gd, qkd, Od, dn_v, DPT, False))


def _sc_layer(Tp_f, Gd_f, Sp, Td, qkp_f, qkd_f, pn_f, pb_f, dn_f, db_f):
    mesh = plsc.VectorSubcoreMesh(core_axis_name="c", subcore_axis_name="s",
                                  num_cores=NC, num_subcores=NS)
    out_type = (
        jax.ShapeDtypeStruct((B * NP, WIDE), jnp.float32),
        jax.ShapeDtypeStruct((B * ND, WIDE), jnp.float32),
    )
    scratch = [
        pltpu.VMEM((ND, H), jnp.float32),
        pltpu.VMEM((ND, WIDE), jnp.float32),
        pltpu.VMEM((PPT, K), jnp.int32),
        pltpu.VMEM((PPT * K,), jnp.int32),
        pltpu.VMEM((DPT * K,), jnp.int32),
        pltpu.VMEM((DPT, K), jnp.int32),
        pltpu.VMEM((2, CH * K, WIDE), jnp.float32),
        pltpu.VMEM((2, CH, H), jnp.float32),
        pltpu.VMEM((2, CH, WIDE), jnp.float32),
        pltpu.SemaphoreType.DMA,
        pltpu.SemaphoreType.DMA,
        pltpu.SemaphoreType.DMA,
        pltpu.SemaphoreType.DMA,
        pltpu.SemaphoreType.DMA,
        pltpu.SemaphoreType.DMA,
    ]
    f = pl.kernel(_sc_body, out_type, mesh=mesh, scratch_types=scratch,
                  compiler_params=pltpu.CompilerParams(
                      needs_layout_passes=False, use_tc_tiling_on_sc=False))
    return f(Tp_f, Gd_f, Sp, Td, qkp_f, qkd_f, pn_f, pb_f, dn_f, db_f)


def kernel(protein_features, drug_features, edge_features_residue_to_atom,
           edge_features_atom_to_residue, params, protein_neighbors,
           drug_neighbors, protein_bonds, drug_bonds, p_hetero_mask,
           d_hetero_mask):
    w = params
    p0, d0, Tp_all, Td_all = _stage0(
        protein_features, drug_features,
        edge_features_residue_to_atom, edge_features_atom_to_residue, w)

    boff = (jnp.arange(B, dtype=jnp.int32) * NP)[:, None, None]
    pn_f = protein_neighbors.astype(jnp.int32).reshape(B * NP, K)
    pb_f = (protein_bonds.astype(jnp.int32) + boff).reshape(B * NP * K)
    dn_f = (drug_neighbors.astype(jnp.int32) + boff).reshape(B * ND * K)
    db_f = drug_bonds.astype(jnp.int32).reshape(B * ND, K)

    up, ud = p0, d0
    for l in range(DEPTH):
        L = w["layers"][l]
        Gd, qkp, Sp, qkd = _pre_tc(up, ud, L)
        Op_f, Od_f = _sc_layer(
            Tp_all[l].reshape(B * NP, WIDE), Gd.reshape(B * NP, WIDE),
            Sp, Td_all[l],
            qkp.reshape(B * NP, H), qkd.reshape(B * ND, H),
            pn_f, pb_f, dn_f, db_f)
        up, ud = _post_tc(Op_f.reshape(B, NP, WIDE), Od_f.reshape(B, ND, WIDE),
                          up, ud, p0, d0, L, add_res=(l == DEPTH - 1))
    return (up, ud)


# speedup vs baseline: 2.1938x; 1.7577x over previous
import functools

import jax
import jax.numpy as jnp
from jax import lax
from jax.experimental import pallas as pl
from jax.experimental.pallas import tpu as pltpu
from jax.experimental.pallas import tpu_sc as plsc

H = 128
EE = 32
WIDE = H + EE
B, NP, ND = 8, 2048, 128
K = 16
DEPTH = 3

NC, NS = 2, 16
NW = NC * NS
PPT = (B * NP) // NW
DPT = (B * ND) // NW
CH = 8
INV_SQRT_H = 1.0 / float(H) ** 0.5
NF = H // 16


def _fullspec(shape):
    nd = len(shape)
    return pl.BlockSpec(shape, lambda b, _n=nd: (0,) * _n)


def _bspec(shape):
    nd = len(shape)
    return pl.BlockSpec((1,) + shape, lambda b, _n=nd: (b,) + (0,) * _n)


def _mm(x, w):
    return jnp.dot(x, w, preferred_element_type=jnp.float32)


def _mmT(x, w):
    return lax.dot_general(x, w, (((1,), (1,)), ((), ())),
                           preferred_element_type=jnp.float32)


def _pre_compute(u, v, Wdcn, Wpcn, WQp, bQp, WKp, WQd, bQd, WKd,
                 Gd_o, qkp_o, Sp_o, qkd_o):
    Gd_o[0, :, 0:H] = _mm(u, Wdcn[...])
    Gd_o[0, :, H:WIDE] = jnp.zeros((NP, EE), jnp.float32)
    Sp_o[0] = _mm(v, Wpcn[...])
    qkp_o[0] = _mmT(_mm(u, WQp[...]) + bQp[...], WKp[...]) * INV_SQRT_H
    qkd_o[0] = _mmT(_mm(v, WQd[...]) + bQd[...], WKd[...]) * INV_SQRT_H


_PRE_OUT = (
    jax.ShapeDtypeStruct((B, NP, WIDE), jnp.float32),
    jax.ShapeDtypeStruct((B, NP, H), jnp.float32),
    jax.ShapeDtypeStruct((B, ND, H), jnp.float32),
    jax.ShapeDtypeStruct((B, ND, H), jnp.float32),
)
_PRE_SPECS = (_bspec((NP, WIDE)), _bspec((NP, H)), _bspec((ND, H)), _bspec((ND, H)))


def _pre_w(L):
    return (L["Wdc"]["W"][:H], L["Wpc"]["W"][:H],
            L["WQp"]["W"], L["WQp"]["b"], L["WKp"]["W"],
            L["WQd"]["W"], L["WQd"]["b"], L["WKd"]["W"])


def _stage0_body(pf, df, er2a, ea2r,
                 Wp1, bp1, Wp2, bp2, Wd1, bd1, Wd2, bd2,
                 Wpb, bpb, Wdb, bdb, Wpce, bpc, Wdce, bdc,
                 Wdcn, Wpcn, WQp, bQp, WKp, WQd, bQd, WKd,
                 p0_o, d0_o, Tp_o, Td_o, Gd_o, qkp_o, Sp_o, qkd_o):
    p0 = _mm(jax.nn.relu(_mm(pf[0], Wp1[...]) + bp1[...]), Wp2[...]) + bp2[...]
    d0 = _mm(jax.nn.relu(_mm(df[0], Wd1[...]) + bd1[...]), Wd2[...]) + bd2[...]
    p0_o[0] = p0
    d0_o[0] = d0
    ep = _mm(ea2r[0], Wpb[...]) + bpb[...]
    ed = _mm(er2a[0], Wdb[...]) + bdb[...]
    for l in range(DEPTH):
        Tp_o[l, 0, :, 0:H] = _mm(ep, Wpce[l]) + bpc[l]
        Tp_o[l, 0, :, H:WIDE] = ep
        Td_o[l, 0, :, 0:H] = _mm(ed, Wdce[l]) + bdc[l]
        Td_o[l, 0, :, H:WIDE] = ed
    _pre_compute(p0, d0, Wdcn, Wpcn, WQp, bQp, WKp, WQd, bQd, WKd,
                 Gd_o, qkp_o, Sp_o, qkd_o)


def _stage0(pf, df, er2a, ea2r, w):
    out_shape = (
        jax.ShapeDtypeStruct((B, NP, H), jnp.float32),
        jax.ShapeDtypeStruct((B, ND, H), jnp.float32),
        jax.ShapeDtypeStruct((DEPTH, B, NP, WIDE), jnp.float32),
        jax.ShapeDtypeStruct((DEPTH, B, ND, WIDE), jnp.float32),
    ) + _PRE_OUT
    ins = (pf, df, er2a, ea2r,
           w["Wu_prot1"]["W"], w["Wu_prot1"]["b"], w["Wu_prot2"]["W"], w["Wu_prot2"]["b"],
           w["Wu_drug1"]["W"], w["Wu_drug1"]["b"], w["Wu_drug2"]["W"], w["Wu_drug2"]["b"],
           w["pb_emb"]["W"], w["pb_emb"]["b"], w["db_emb"]["W"], w["db_emb"]["b"],
           jnp.stack([L["Wpc"]["W"][H:] for L in w["layers"]]),
           jnp.stack([L["Wpc"]["b"] for L in w["layers"]]),
           jnp.stack([L["Wdc"]["W"][H:] for L in w["layers"]]),
           jnp.stack([L["Wdc"]["b"] for L in w["layers"]])) + _pre_w(w["layers"][0])
    in_specs = [_bspec(x.shape[1:]) for x in (pf, df, er2a, ea2r)]
    in_specs += [_fullspec(x.shape) for x in ins[4:]]
    out_specs = (
        _bspec((NP, H)), _bspec((ND, H)),
        pl.BlockSpec((DEPTH, 1, NP, WIDE), lambda b: (0, b, 0, 0)),
        pl.BlockSpec((DEPTH, 1, ND, WIDE), lambda b: (0, b, 0, 0)),
    ) + _PRE_SPECS
    return pl.pallas_call(
        _stage0_body, grid=(B,), out_shape=out_shape,
        in_specs=in_specs, out_specs=out_specs)(*ins)


def _post_body(Op, Od, up, ud, p0, d0,
               Wp, bp, Wpup, bpup, Wd, bd, Wdup, bdup,
               *rest, add_res):
    hp = jax.nn.relu(_mm(Op[0], Wp[...]) + bp[...])
    hd = jax.nn.relu(_mm(Od[0], Wd[...]) + bd[...])
    un = up[0] + _mm(hp, Wpup[...]) + bpup[...]
    vn = ud[0] + _mm(hd, Wdup[...]) + bdup[...]
    if add_res:
        un = un + p0[0]
        vn = vn + d0[0]
        up_o, ud_o = rest
        up_o[0] = un
        ud_o[0] = vn
    else:
        (Wdcn, Wpcn, WQp, bQp, WKp, WQd, bQd, WKd,
         up_o, ud_o, Gd_o, qkp_o, Sp_o, qkd_o) = rest
        up_o[0] = un
        ud_o[0] = vn
        _pre_compute(un, vn, Wdcn, Wpcn, WQp, bQp, WKp, WQd, bQd, WKd,
                     Gd_o, qkp_o, Sp_o, qkd_o)


def _post_tc(Op, Od, up, ud, p0, d0, L, L_next):
    out_shape = (
        jax.ShapeDtypeStruct((B, NP, H), jnp.float32),
        jax.ShapeDtypeStruct((B, ND, H), jnp.float32),
    )
    ins = (Op, Od, up, ud, p0, d0,
           L["Wp"]["W"], L["Wp"]["b"], L["Wpup"]["W"], L["Wpup"]["b"],
           L["Wd"]["W"], L["Wd"]["b"], L["Wdup"]["W"], L["Wdup"]["b"])
    out_specs = (_bspec((NP, H)), _bspec((ND, H)))
    if L_next is not None:
        ins = ins + _pre_w(L_next)
        out_shape = out_shape + _PRE_OUT
        out_specs = out_specs + _PRE_SPECS
    in_specs = [_bspec((NP, WIDE)), _bspec((ND, WIDE)),
                _bspec((NP, H)), _bspec((ND, H)),
                _bspec((NP, H)), _bspec((ND, H))]
    in_specs += [_fullspec(x.shape) for x in ins[6:]]
    return pl.pallas_call(
        functools.partial(_post_body, add_res=(L_next is None)),
        grid=(B,), out_shape=out_shape,
        in_specs=in_specs, out_specs=out_specs)(*ins)


def _attend_node(n, sl, qk_v, out_v, idx_row, nei_ld, eproj_ld, eraw_ld):
    qk = [qk_v[sl, n, pl.ds(16 * f, 16)] for f in range(NF)]
    accs = [jnp.zeros((16,), jnp.float32) for _ in range(NF + 2)]
    dsum = jnp.zeros((16,), jnp.float32)
    for k in range(K):
        ctx = (idx_row[k], n * K + k)
        ms = []
        acc = None
        for f in range(NF):
            x = nei_ld(ctx, f) + eproj_ld(ctx, f)
            m = jnp.maximum(x, 0.1 * x)
            ms.append(m)
            t = qk[f] * m
            acc = t if acc is None else acc + t
        s = jnp.sum(acc)
        ev = jnp.exp(jnp.zeros((16,), jnp.float32) + s)
        dsum = dsum + ev
        for f in range(NF):
            accs[f] = accs[f] + ev * ms[f]
        for f in range(2):
            accs[NF + f] = accs[NF + f] + ev * eraw_ld(ctx, f)
    inv = 1.0 / (dsum + 1e-6)
    for f in range(NF + 2):
        out_v[sl, n, pl.ds(16 * f, 16)] = accs[f] * inv


def _pipeline(nch, fetch_start, fetch_wait, out_start, out_wait, compute):
    fetch_start(0, 0)

    def step(c, carry):
        s = c % 2
        for t in (0, 1):
            @pl.when(s == t)
            def _(t=t):
                @pl.when(c + 1 < nch)
                def _():
                    fetch_start(c + 1, 1 - t)
                fetch_wait(c, t)

                @pl.when(c >= 2)
                def _():
                    out_wait(c - 2, t)
        compute(c, s)
        for t in (0, 1):
            @pl.when(s == t)
            def _(t=t):
                out_start(c, t)
        return carry

    lax.fori_loop(0, nch, step, 0)
    for c in (nch - 2, nch - 1):
        out_wait(c, c % 2)


def _sc_body(Tp, Gd, Sp, Td, qkp, qkd, pn, pb, dn, db, Op, Od,
             small_p, small_d, pn_v, pb_v, dn_v, db_v,
             buf, qk_v, out_v, sg0, sg1, sq0, sq1, so0, so1):
    cid = lax.axis_index("c")
    sid = lax.axis_index("s")
    wid = sid * NC + cid
    bat = wid // (NW // B)
    sg = (sg0, sg1)
    sq = (sq0, sq1)
    so = (so0, so1)

    pltpu.sync_copy(Sp.at[bat], small_p)
    pltpu.sync_copy(Td.at[bat], small_d)
    pltpu.sync_copy(pn.at[pl.ds(wid * PPT, PPT)], pn_v)
    pltpu.sync_copy(pb.at[pl.ds(wid * PPT * K, PPT * K)], pb_v)
    pltpu.sync_copy(dn.at[pl.ds(wid * DPT * K, DPT * K)], dn_v)
    pltpu.sync_copy(db.at[pl.ds(wid * DPT, DPT)], db_v)

    def make_phase(tab, qk_hbm, out_hbm, idx_v, per_tile, is_protein, unroll=1):
        def g_copy(c, t):
            return pltpu.make_async_copy(
                tab.at[idx_v.at[pl.ds(c * CH * K, CH * K)]], buf.at[t], sg[t])

        def q_copy(c, t):
            gbase = wid * per_tile + c * CH
            return pltpu.make_async_copy(
                qk_hbm.at[pl.ds(gbase, CH)], qk_v.at[t], sq[t])

        def o_copy(c, t):
            gbase = wid * per_tile + c * CH
            return pltpu.make_async_copy(
                out_v.at[t], out_hbm.at[pl.ds(gbase, CH)], so[t])

        def fetch_start(c, t):
            g_copy(c, t).start()
            q_copy(c, t).start()

        def fetch_wait(c, t):
            g_copy(c, t).wait()
            q_copy(c, t).wait()

        def out_start(c, t):
            o_copy(c, t).start()

        def out_wait(c, t):
            o_copy(c, t).wait()

        def compute(c, s):
            @plsc.parallel_loop(0, CH, unroll=unroll)
            def node(n):
                nl = c * CH + n
                if is_protein:
                    _attend_node(
                        n, s, qk_v, out_v, pn_v[nl, :],
                        lambda ctx, f: small_p[ctx[0], pl.ds(16 * f, 16)],
                        lambda ctx, f: buf[s, ctx[1], pl.ds(16 * f, 16)],
                        lambda ctx, f: buf[s, ctx[1], pl.ds(H + 16 * f, 16)])
                else:
                    _attend_node(
                        n, s, qk_v, out_v, db_v[nl, :],
                        lambda ctx, f: buf[s, ctx[1], pl.ds(16 * f, 16)],
                        lambda ctx, f: small_d[ctx[0], pl.ds(16 * f, 16)],
                        lambda ctx, f: small_d[ctx[0], pl.ds(H + 16 * f, 16)])

        return fetch_start, fetch_wait, out_start, out_wait, compute

    _pipeline(PPT // CH, *make_phase(Tp, qkp, Op, pb_v, PPT, True, unroll=1))
    _pipeline(DPT // CH, *make_phase(Gd, qkd, Od, dn_v, DPT, False))


def _sc_layer(Tp_f, Gd_f, Sp, Td, qkp_f, qkd_f, pn_f, pb_f, dn_f, db_f):
    mesh = plsc.VectorSubcoreMesh(core_axis_name="c", subcore_axis_name="s",
                                  num_cores=NC, num_subcores=NS)
    out_type = (
        jax.ShapeDtypeStruct((B * NP, WIDE), jnp.float32),
        jax.ShapeDtypeStruct((B * ND, WIDE), jnp.float32),
    )
    scratch = [
        pltpu.VMEM((ND, H), jnp.float32),
        pltpu.VMEM((ND, WIDE), jnp.float32),
        pltpu.VMEM((PPT, K), jnp.int32),
        pltpu.VMEM((PPT * K,), jnp.int32),
        pltpu.VMEM((DPT * K,), jnp.int32),
        pltpu.VMEM((DPT, K), jnp.int32),
        pltpu.VMEM((2, CH * K, WIDE), jnp.float32),
        pltpu.VMEM((2, CH, H), jnp.float32),
        pltpu.VMEM((2, CH, WIDE), jnp.float32),
        pltpu.SemaphoreType.DMA,
        pltpu.SemaphoreType.DMA,
        pltpu.SemaphoreType.DMA,
        pltpu.SemaphoreType.DMA,
        pltpu.SemaphoreType.DMA,
        pltpu.SemaphoreType.DMA,
    ]
    f = pl.kernel(_sc_body, out_type, mesh=mesh, scratch_types=scratch,
                  compiler_params=pltpu.CompilerParams(
                      needs_layout_passes=False, use_tc_tiling_on_sc=False))
    return f(Tp_f, Gd_f, Sp, Td, qkp_f, qkd_f, pn_f, pb_f, dn_f, db_f)


def kernel(protein_features, drug_features, edge_features_residue_to_atom,
           edge_features_atom_to_residue, params, protein_neighbors,
           drug_neighbors, protein_bonds, drug_bonds, p_hetero_mask,
           d_hetero_mask):
    w = params
    p0, d0, Tp_all, Td_all, Gd, qkp, Sp, qkd = _stage0(
        protein_features, drug_features,
        edge_features_residue_to_atom, edge_features_atom_to_residue, w)

    boff = (jnp.arange(B, dtype=jnp.int32) * NP)[:, None, None]
    pn_f = protein_neighbors.astype(jnp.int32).reshape(B * NP, K)
    pb_f = (protein_bonds.astype(jnp.int32) + boff).reshape(B * NP * K)
    dn_f = (drug_neighbors.astype(jnp.int32) + boff).reshape(B * ND * K)
    db_f = drug_bonds.astype(jnp.int32).reshape(B * ND, K)

    up, ud = p0, d0
    for l in range(DEPTH):
        L = w["layers"][l]
        L_next = w["layers"][l + 1] if l + 1 < DEPTH else None
        Op_f, Od_f = _sc_layer(
            Tp_all[l].reshape(B * NP, WIDE), Gd.reshape(B * NP, WIDE),
            Sp, Td_all[l],
            qkp.reshape(B * NP, H), qkd.reshape(B * ND, H),
            pn_f, pb_f, dn_f, db_f)
        outs = _post_tc(Op_f.reshape(B, NP, WIDE), Od_f.reshape(B, ND, WIDE),
                        up, ud, p0, d0, L, L_next)
        if L_next is not None:
            up, ud, Gd, qkp, Sp, qkd = outs
        else:
            up, ud = outs
    return (up, ud)


# speedup vs baseline: 2.2263x; 1.0148x over previous
import functools

import jax
import jax.numpy as jnp
from jax import lax
from jax.experimental import pallas as pl
from jax.experimental.pallas import tpu as pltpu
from jax.experimental.pallas import tpu_sc as plsc

H = 128
EE = 32
WIDE = H + EE
B, NP, ND = 8, 2048, 128
K = 16
DEPTH = 3

NC, NS = 2, 16
NW = NC * NS
PPT = (B * NP) // NW
DPT = (B * ND) // NW
CH = 8
INV_SQRT_H = 1.0 / float(H) ** 0.5
NF = H // 16


def _fullspec(shape):
    nd = len(shape)
    return pl.BlockSpec(shape, lambda b, _n=nd: (0,) * _n)


def _bspec(shape):
    nd = len(shape)
    return pl.BlockSpec((1,) + shape, lambda b, _n=nd: (b,) + (0,) * _n)


def _mm(x, w):
    return jnp.dot(x.astype(jnp.bfloat16), w[...].astype(jnp.bfloat16),
                   preferred_element_type=jnp.float32)


def _mmT(x, w):
    return lax.dot_general(x.astype(jnp.bfloat16), w[...].astype(jnp.bfloat16),
                           (((1,), (1,)), ((), ())),
                           preferred_element_type=jnp.float32)


def _pre_compute(u, v, Wdcn, Wpcn, WQp, bQp, WKp, WQd, bQd, WKd,
                 Gd_o, qkp_o, Sp_o, qkd_o):
    Gd_o[0, :, 0:H] = _mm(u, Wdcn[...])
    Sp_o[0] = _mm(v, Wpcn[...])
    qkp_o[0] = _mmT(_mm(u, WQp[...]) + bQp[...], WKp[...]) * INV_SQRT_H
    qkd_o[0] = _mmT(_mm(v, WQd[...]) + bQd[...], WKd[...]) * INV_SQRT_H


_PRE_OUT = (
    jax.ShapeDtypeStruct((B, NP, WIDE), jnp.float32),
    jax.ShapeDtypeStruct((B, NP, H), jnp.float32),
    jax.ShapeDtypeStruct((B, ND, H), jnp.float32),
    jax.ShapeDtypeStruct((B, ND, H), jnp.float32),
)
_PRE_SPECS = (_bspec((NP, WIDE)), _bspec((NP, H)), _bspec((ND, H)), _bspec((ND, H)))


def _pre_w(L):
    return (L["Wdc"]["W"][:H], L["Wpc"]["W"][:H],
            L["WQp"]["W"], L["WQp"]["b"], L["WKp"]["W"],
            L["WQd"]["W"], L["WQd"]["b"], L["WKd"]["W"])


def _stage0_body(pf, df, er2a, ea2r,
                 Wp1, bp1, Wp2, bp2, Wd1, bd1, Wd2, bd2,
                 Wpb, bpb, Wdb, bdb, Wpce, bpc, Wdce, bdc,
                 Wdcn, Wpcn, WQp, bQp, WKp, WQd, bQd, WKd,
                 p0_o, d0_o, Tp_o, Td_o, Gd_o, qkp_o, Sp_o, qkd_o):
    p0 = _mm(jax.nn.relu(_mm(pf[0], Wp1[...]) + bp1[...]), Wp2[...]) + bp2[...]
    d0 = _mm(jax.nn.relu(_mm(df[0], Wd1[...]) + bd1[...]), Wd2[...]) + bd2[...]
    p0_o[0] = p0
    d0_o[0] = d0
    ep = _mm(ea2r[0], Wpb[...]) + bpb[...]
    ed = _mm(er2a[0], Wdb[...]) + bdb[...]
    for l in range(DEPTH):
        Tp_o[l, 0, :, 0:H] = _mm(ep, Wpce[l]) + bpc[l]
        Tp_o[l, 0, :, H:WIDE] = ep
        Td_o[l, 0, :, 0:H] = _mm(ed, Wdce[l]) + bdc[l]
        Td_o[l, 0, :, H:WIDE] = ed
    _pre_compute(p0, d0, Wdcn, Wpcn, WQp, bQp, WKp, WQd, bQd, WKd,
                 Gd_o, qkp_o, Sp_o, qkd_o)


def _stage0(pf, df, er2a, ea2r, w):
    out_shape = (
        jax.ShapeDtypeStruct((B, NP, H), jnp.float32),
        jax.ShapeDtypeStruct((B, ND, H), jnp.float32),
        jax.ShapeDtypeStruct((DEPTH, B, NP, WIDE), jnp.float32),
        jax.ShapeDtypeStruct((DEPTH, B, ND, WIDE), jnp.float32),
    ) + _PRE_OUT
    ins = (pf, df, er2a, ea2r,
           w["Wu_prot1"]["W"], w["Wu_prot1"]["b"], w["Wu_prot2"]["W"], w["Wu_prot2"]["b"],
           w["Wu_drug1"]["W"], w["Wu_drug1"]["b"], w["Wu_drug2"]["W"], w["Wu_drug2"]["b"],
           w["pb_emb"]["W"], w["pb_emb"]["b"], w["db_emb"]["W"], w["db_emb"]["b"],
           jnp.stack([L["Wpc"]["W"][H:] for L in w["layers"]]),
           jnp.stack([L["Wpc"]["b"] for L in w["layers"]]),
           jnp.stack([L["Wdc"]["W"][H:] for L in w["layers"]]),
           jnp.stack([L["Wdc"]["b"] for L in w["layers"]])) + _pre_w(w["layers"][0])
    in_specs = [_bspec(x.shape[1:]) for x in (pf, df, er2a, ea2r)]
    in_specs += [_fullspec(x.shape) for x in ins[4:]]
    out_specs = (
        _bspec((NP, H)), _bspec((ND, H)),
        pl.BlockSpec((DEPTH, 1, NP, WIDE), lambda b: (0, b, 0, 0)),
        pl.BlockSpec((DEPTH, 1, ND, WIDE), lambda b: (0, b, 0, 0)),
    ) + _PRE_SPECS
    return pl.pallas_call(
        _stage0_body, grid=(B,), out_shape=out_shape,
        in_specs=in_specs, out_specs=out_specs)(*ins)


def _post_body(Op, Od, up, ud, p0, d0,
               Wp, bp, Wpup, bpup, Wd, bd, Wdup, bdup,
               *rest, add_res):
    hp = jax.nn.relu(_mm(Op[0], Wp[...]) + bp[...])
    hd = jax.nn.relu(_mm(Od[0], Wd[...]) + bd[...])
    un = up[0] + _mm(hp, Wpup[...]) + bpup[...]
    vn = ud[0] + _mm(hd, Wdup[...]) + bdup[...]
    if add_res:
        un = un + p0[0]
        vn = vn + d0[0]
        up_o, ud_o = rest
        up_o[0] = un
        ud_o[0] = vn
    else:
        (Wdcn, Wpcn, WQp, bQp, WKp, WQd, bQd, WKd,
         up_o, ud_o, Gd_o, qkp_o, Sp_o, qkd_o) = rest
        up_o[0] = un
        ud_o[0] = vn
        _pre_compute(un, vn, Wdcn, Wpcn, WQp, bQp, WKp, WQd, bQd, WKd,
                     Gd_o, qkp_o, Sp_o, qkd_o)


def _post_tc(Op, Od, up, ud, p0, d0, L, L_next):
    out_shape = (
        jax.ShapeDtypeStruct((B, NP, H), jnp.float32),
        jax.ShapeDtypeStruct((B, ND, H), jnp.float32),
    )
    ins = (Op, Od, up, ud, p0, d0,
           L["Wp"]["W"], L["Wp"]["b"], L["Wpup"]["W"], L["Wpup"]["b"],
           L["Wd"]["W"], L["Wd"]["b"], L["Wdup"]["W"], L["Wdup"]["b"])
    out_specs = (_bspec((NP, H)), _bspec((ND, H)))
    if L_next is not None:
        ins = ins + _pre_w(L_next)
        out_shape = out_shape + _PRE_OUT
        out_specs = out_specs + _PRE_SPECS
    in_specs = [_bspec((NP, WIDE)), _bspec((ND, WIDE)),
                _bspec((NP, H)), _bspec((ND, H)),
                _bspec((NP, H)), _bspec((ND, H))]
    in_specs += [_fullspec(x.shape) for x in ins[6:]]
    return pl.pallas_call(
        functools.partial(_post_body, add_res=(L_next is None)),
        grid=(B,), out_shape=out_shape,
        in_specs=in_specs, out_specs=out_specs)(*ins)


def _attend_node(n, sl, qk_v, out_v, idx_row, nei_ld, eproj_ld, eraw_ld):
    qk = [qk_v[sl, n, pl.ds(16 * f, 16)] for f in range(NF)]
    accs = [jnp.zeros((16,), jnp.float32) for _ in range(NF + 2)]
    dsum = jnp.zeros((16,), jnp.float32)
    for k in range(K):
        ctx = (idx_row[k], n * K + k)
        ms = []
        acc = None
        for f in range(NF):
            x = nei_ld(ctx, f) + eproj_ld(ctx, f)
            m = jnp.maximum(x, 0.1 * x)
            ms.append(m)
            t = qk[f] * m
            acc = t if acc is None else acc + t
        s = jnp.sum(acc)
        ev = jnp.exp(jnp.zeros((16,), jnp.float32) + s)
        dsum = dsum + ev
        for f in range(NF):
            accs[f] = accs[f] + ev * ms[f]
        for f in range(2):
            accs[NF + f] = accs[NF + f] + ev * eraw_ld(ctx, f)
    inv = 1.0 / (dsum + 1e-6)
    for f in range(NF + 2):
        out_v[sl, n, pl.ds(16 * f, 16)] = accs[f] * inv


def _pipeline(nch, fetch_start, fetch_wait, out_start, out_wait, compute):
    fetch_start(0, 0)

    def step(c, carry):
        s = c % 2
        for t in (0, 1):
            @pl.when(s == t)
            def _(t=t):
                @pl.when(c + 1 < nch)
                def _():
                    fetch_start(c + 1, 1 - t)
                fetch_wait(c, t)

                @pl.when(c >= 2)
                def _():
                    out_wait(c - 2, t)
        compute(c, s)
        for t in (0, 1):
            @pl.when(s == t)
            def _(t=t):
                out_start(c, t)
        return carry

    lax.fori_loop(0, nch, step, 0)
    for c in (nch - 2, nch - 1):
        out_wait(c, c % 2)


def _sc_body(Tp, Gd, Sp, Td, qkp, qkd, pn, pb, dn, db, Op, Od,
             small_p, small_d, pn_v, pb_v, dn_v, db_v,
             buf, qk_v, out_v, sg0, sg1, sq0, sq1, so0, so1):
    cid = lax.axis_index("c")
    sid = lax.axis_index("s")
    wid = sid * NC + cid
    bat = wid // (NW // B)
    sg = (sg0, sg1)
    sq = (sq0, sq1)
    so = (so0, so1)

    prologue = (
        pltpu.make_async_copy(Sp.at[bat], small_p, sg0),
        pltpu.make_async_copy(Td.at[bat], small_d, sg1),
        pltpu.make_async_copy(pn.at[pl.ds(wid * PPT, PPT)], pn_v, sq0),
        pltpu.make_async_copy(pb.at[pl.ds(wid * PPT * K, PPT * K)], pb_v, sq1),
        pltpu.make_async_copy(dn.at[pl.ds(wid * DPT * K, DPT * K)], dn_v, so0),
        pltpu.make_async_copy(db.at[pl.ds(wid * DPT, DPT)], db_v, so1),
    )
    for cp in prologue:
        cp.start()
    for cp in prologue:
        cp.wait()

    def make_phase(tab, qk_hbm, out_hbm, idx_v, per_tile, is_protein, unroll=1):
        def g_copy(c, t):
            return pltpu.make_async_copy(
                tab.at[idx_v.at[pl.ds(c * CH * K, CH * K)]], buf.at[t], sg[t])

        def q_copy(c, t):
            gbase = wid * per_tile + c * CH
            return pltpu.make_async_copy(
                qk_hbm.at[pl.ds(gbase, CH)], qk_v.at[t], sq[t])

        def o_copy(c, t):
            gbase = wid * per_tile + c * CH
            return pltpu.make_async_copy(
                out_v.at[t], out_hbm.at[pl.ds(gbase, CH)], so[t])

        def fetch_start(c, t):
            g_copy(c, t).start()
            q_copy(c, t).start()

        def fetch_wait(c, t):
            g_copy(c, t).wait()
            q_copy(c, t).wait()

        def out_start(c, t):
            o_copy(c, t).start()

        def out_wait(c, t):
            o_copy(c, t).wait()

        def compute(c, s):
            @plsc.parallel_loop(0, CH, unroll=unroll)
            def node(n):
                nl = c * CH + n
                if is_protein:
                    _attend_node(
                        n, s, qk_v, out_v, pn_v[nl, :],
                        lambda ctx, f: small_p[ctx[0], pl.ds(16 * f, 16)],
                        lambda ctx, f: buf[s, ctx[1], pl.ds(16 * f, 16)],
                        lambda ctx, f: buf[s, ctx[1], pl.ds(H + 16 * f, 16)])
                else:
                    _attend_node(
                        n, s, qk_v, out_v, db_v[nl, :],
                        lambda ctx, f: buf[s, ctx[1], pl.ds(16 * f, 16)],
                        lambda ctx, f: small_d[ctx[0], pl.ds(16 * f, 16)],
                        lambda ctx, f: small_d[ctx[0], pl.ds(H + 16 * f, 16)])

        return fetch_start, fetch_wait, out_start, out_wait, compute

    _pipeline(PPT // CH, *make_phase(Tp, qkp, Op, pb_v, PPT, True, unroll=1))
    _pipeline(DPT // CH, *make_phase(Gd, qkd, Od, dn_v, DPT, False))


def _sc_layer(Tp_f, Gd_f, Sp, Td, qkp_f, qkd_f, pn_f, pb_f, dn_f, db_f):
    mesh = plsc.VectorSubcoreMesh(core_axis_name="c", subcore_axis_name="s",
                                  num_cores=NC, num_subcores=NS)
    out_type = (
        jax.ShapeDtypeStruct((B * NP, WIDE), jnp.float32),
        jax.ShapeDtypeStruct((B * ND, WIDE), jnp.float32),
    )
    scratch = [
        pltpu.VMEM((ND, H), jnp.float32),
        pltpu.VMEM((ND, WIDE), jnp.float32),
        pltpu.VMEM((PPT, K), jnp.int32),
        pltpu.VMEM((PPT * K,), jnp.int32),
        pltpu.VMEM((DPT * K,), jnp.int32),
        pltpu.VMEM((DPT, K), jnp.int32),
        pltpu.VMEM((2, CH * K, WIDE), jnp.float32),
        pltpu.VMEM((2, CH, H), jnp.float32),
        pltpu.VMEM((2, CH, WIDE), jnp.float32),
        pltpu.SemaphoreType.DMA,
        pltpu.SemaphoreType.DMA,
        pltpu.SemaphoreType.DMA,
        pltpu.SemaphoreType.DMA,
        pltpu.SemaphoreType.DMA,
        pltpu.SemaphoreType.DMA,
    ]
    f = pl.kernel(_sc_body, out_type, mesh=mesh, scratch_types=scratch,
                  compiler_params=pltpu.CompilerParams(
                      needs_layout_passes=False, use_tc_tiling_on_sc=False))
    return f(Tp_f, Gd_f, Sp, Td, qkp_f, qkd_f, pn_f, pb_f, dn_f, db_f)


def kernel(protein_features, drug_features, edge_features_residue_to_atom,
           edge_features_atom_to_residue, params, protein_neighbors,
           drug_neighbors, protein_bonds, drug_bonds, p_hetero_mask,
           d_hetero_mask):
    w = params
    p0, d0, Tp_all, Td_all, Gd, qkp, Sp, qkd = _stage0(
        protein_features, drug_features,
        edge_features_residue_to_atom, edge_features_atom_to_residue, w)

    boff = (jnp.arange(B, dtype=jnp.int32) * NP)[:, None, None]
    pn_f = protein_neighbors.astype(jnp.int32).reshape(B * NP, K)
    pb_f = (protein_bonds.astype(jnp.int32) + boff).reshape(B * NP * K)
    dn_f = (drug_neighbors.astype(jnp.int32) + boff).reshape(B * ND * K)
    db_f = drug_bonds.astype(jnp.int32).reshape(B * ND, K)

    up, ud = p0, d0
    for l in range(DEPTH):
        L = w["layers"][l]
        L_next = w["layers"][l + 1] if l + 1 < DEPTH else None
        Op_f, Od_f = _sc_layer(
            Tp_all[l].reshape(B * NP, WIDE), Gd.reshape(B * NP, WIDE),
            Sp, Td_all[l],
            qkp.reshape(B * NP, H), qkd.reshape(B * ND, H),
            pn_f, pb_f, dn_f, db_f)
        outs = _post_tc(Op_f.reshape(B, NP, WIDE), Od_f.reshape(B, ND, WIDE),
                        up, ud, p0, d0, L, L_next)
        if L_next is not None:
            up, ud, Gd, qkp, Sp, qkd = outs
        else:
            up, ud = outs
    return (up, ud)
